# baseline (speedup 1.0000x reference)
import numpy as np
import jax
import jax.numpy as jnp
from jax.experimental import pallas as pl
from jax.experimental.pallas import tpu as pltpu

C = 768
HEADS = 12
DEPTH = 12
HD = C // HEADS
NT = 257
NP = 264
NS = 256
CH = 3351
PK = 1536
SCALE = HD ** -0.5
NEG = -1e30
EPS = 1e-5
CT = 384
MT = -(-CH // CT)
NSTG = 4 + MT

_VMEM_LIMIT = 50 * 1024 * 1024


def _build_consts():
    g = np.arange(16)
    xx = np.repeat(g, 16)
    yy = np.tile(g, 16)
    win = (np.abs(xx[:, None] - xx[None, :]) <= 3) & (np.abs(yy[:, None] - yy[None, :]) <= 3)
    loc_bias = np.where(win, 0.0, NEG).astype(np.float32)
    nx = np.minimum(xx + 3, 15) - np.maximum(xx - 3, 0) + 1
    ny = np.minimum(yy + 3, 15) - np.maximum(yy - 3, 0) + 1
    npad = (49 - nx * ny).astype(np.float32).reshape(NS, 1)
    gb = np.zeros((1, NP), np.float32)
    gb[0, NT:] = NEG
    return loc_bias, npad, gb


_LOC_BIAS_NP, _NPAD_NP, _GLB_BIAS_NP = _build_consts()


def _ln(x, w, b):
    m = jnp.mean(x, axis=-1, keepdims=True)
    xc = x - m
    v = jnp.mean(xc * xc, axis=-1, keepdims=True)
    return xc * jax.lax.rsqrt(v + EPS) * w + b


def _mmt(a, b):
    return jax.lax.dot_general(a, b, (((1,), (1,)), ((), ())),
                               preferred_element_type=jnp.float32)


def _mm(a, b):
    return jax.lax.dot_general(a, b, (((1,), (0,)), ((), ())),
                               preferred_element_type=jnp.float32)


def _gelu(x):
    return 0.5 * x * (1.0 + jax.lax.erf(x * (2.0 ** -0.5)))


def _embed_body(p_ref, w_ref, cb_ref, lnw_ref, lnb_ref, pos_ref, o_ref):
    x = p_ref[...].reshape(2 * NS, PK)
    pe = _mmt(x, w_ref[...]) + cb_ref[...]
    pe = _ln(pe, lnw_ref[...], lnb_ref[...])
    o_ref[...] = pe.reshape(2, NS, C) + pos_ref[...][None]


def _embed(patches, wpe, cb, lnw, lnb, pos):
    full = lambda a: pl.BlockSpec(a.shape, lambda c: (0,) * a.ndim)
    return pl.pallas_call(
        _embed_body,
        grid=(2,),
        in_specs=[pl.BlockSpec((2, NS, PK), lambda c: (c, 0, 0)),
                  full(wpe), full(cb), full(lnw), full(lnb), full(pos)],
        out_specs=pl.BlockSpec((2, NS, C), lambda c: (c, 0, 0)),
        out_shape=jax.ShapeDtypeStruct((4, NS, C), jnp.float32),
        compiler_params=pltpu.CompilerParams(
            dimension_semantics=("parallel",),
            vmem_limit_bytes=_VMEM_LIMIT),
        name="d4rt_embed",
    )(patches, wpe, cb, lnw, lnb, pos)


MB = 4 * NP


def _mega_body(h_ref, n1w, n1b, n2w, n2b, lqw, lpw, lpb, gqw, gqb, gow, gob,
               f1w, f1b, f2w, f2b, fnw, fnb, lb_ref, npad_ref, gb_ref,
               o_ref, h_s, ln_s, qkv_s, ao_s):
    l = pl.program_id(0)
    t = pl.program_id(1)
    even = (l % 2) == 0
    bf16 = jnp.bfloat16

    @pl.when((l == 0) & (t == 0))
    def _():
        h_s[...] = h_ref[...]

    @pl.when(t == 0)
    def _():
        ln_s[...] = _ln(h_s[...].reshape(MB, C), n1w[0], n1b[0]).astype(bf16)

    @pl.when(t < 3)
    def _():
        col = pl.ds(t * C, C)

        @pl.when(even)
        def _():
            qkv_s[:, col] = _mmt(ln_s[...], lqw[0].astype(bf16)).astype(bf16)

        @pl.when(~even)
        def _():
            qkv_s[:, col] = (_mmt(ln_s[...], gqw[0].astype(bf16))
                             + gqb[0]).astype(bf16)

    @pl.when(t == 3)
    def _():
        @pl.when(even)
        def _():
            for b in range(4):
                r0 = b * NP
                for hh in range(HEADS):
                    c0 = hh * HD
                    q = qkv_s[r0:r0 + NS, c0:c0 + HD]
                    k = qkv_s[r0:r0 + NS, C + c0:C + c0 + HD]
                    v = qkv_s[r0:r0 + NS, 2 * C + c0:2 * C + c0 + HD]
                    s = _mmt(q, k) * SCALE + lb_ref[...]
                    m = jnp.maximum(jnp.max(s, axis=-1, keepdims=True), 0.0)
                    e = jnp.exp(s - m)
                    den = (jnp.sum(e, axis=-1, keepdims=True)
                           + npad_ref[...] * jnp.exp(-m))
                    ao_s[r0:r0 + NS, c0:c0 + HD] = _mm(
                        (e / den).astype(bf16), v).astype(bf16)
            out = _mmt(ao_s[...], lpw[0].astype(bf16)) + lpb[0]
            o2 = out.reshape(4, NP, C)
            h_s[:, :NS, :] = h_s[:, :NS, :] + o2[:, :NS, :]

        @pl.when(~even)
        def _():
            for b in range(4):
                r0 = b * NP
                for hh in range(HEADS):
                    c0 = hh * HD
                    q = qkv_s[r0:r0 + NP, c0:c0 + HD]
                    k = qkv_s[r0:r0 + NP, C + c0:C + c0 + HD]
                    v = qkv_s[r0:r0 + NP, 2 * C + c0:2 * C + c0 + HD]
                    s = _mmt(q, k) * SCALE + gb_ref[...]
                    m = jnp.max(s, axis=-1, keepdims=True)
                    e = jnp.exp(s - m)
                    p = e / jnp.sum(e, axis=-1, keepdims=True)
                    ao_s[r0:r0 + NP, c0:c0 + HD] = _mm(
                        p.astype(bf16), v).astype(bf16)
            out = _mmt(ao_s[...], gow[0].astype(bf16)) + gob[0]
            h_s[...] = h_s[...] + out.reshape(4, NP, C)

        ln_s[...] = _ln(h_s[...].reshape(MB, C), n2w[0], n2b[0]).astype(bf16)

    @pl.when(t >= 4)
    def _():
        hcol = _mmt(ln_s[...], f1w[0].astype(bf16)) + f1b[0]
        lane = (t - 4) * CT + jax.lax.broadcasted_iota(jnp.int32, (1, CT), 1)
        valid = lane < CH
        hcol = jnp.where(valid, _gelu(hcol), 0.0).astype(bf16)
        f2wv = jnp.where(valid, f2w[0].astype(bf16), jnp.bfloat16(0.0))
        part = _mmt(hcol, f2wv).reshape(4, NP, C)

        @pl.when(t < NSTG - 1)
        def _():
            h_s[...] = h_s[...] + part

        @pl.when(t == NSTG - 1)
        def _():
            fin = h_s[...].reshape(MB, C) + part.reshape(MB, C) + f2b[0]
            h_s[...] = fin.reshape(4, NP, C)

            @pl.when(l == DEPTH - 1)
            def _():
                o_ref[...] = _ln(fin, fnw[...], fnb[...]).reshape(4, NP, C)


def _mega(h, n1w3, n1b3, n2w3, n2b3, lqw, lpw, lpb3, gqw, gqb3, gow, gob3,
          f1w, f1b3, f2w, f2b3, fnw2, fnb2, lb, npad, gb):
    def qc(t):
        return jnp.minimum(t, 2)

    def mc(t):
        return jnp.clip(t - 4, 0, MT - 1)

    e_ = lambda l: l // 2
    even_ = lambda l: (l % 2) == 0
    lq_c = lambda l, t: jnp.where(even_(l), qc(t), 2)
    gq_c = lambda l, t: jnp.where(even_(l), 2, qc(t))

    in_specs = [
        pl.BlockSpec((4, NP, C), lambda l, t: (0, 0, 0)),
        pl.BlockSpec((1, 1, C), lambda l, t: (l, 0, 0)),
        pl.BlockSpec((1, 1, C), lambda l, t: (l, 0, 0)),
        pl.BlockSpec((1, 1, C), lambda l, t: (l, 0, 0)),
        pl.BlockSpec((1, 1, C), lambda l, t: (l, 0, 0)),
        pl.BlockSpec((1, C, C), lambda l, t: (e_(l), lq_c(l, t), 0)),
        pl.BlockSpec((1, C, C), lambda l, t: (e_(l), 0, 0)),
        pl.BlockSpec((1, 1, C), lambda l, t: (e_(l), 0, 0)),
        pl.BlockSpec((1, C, C), lambda l, t: (e_(l), gq_c(l, t), 0)),
        pl.BlockSpec((1, 1, C), lambda l, t: (e_(l), 0, gq_c(l, t))),
        pl.BlockSpec((1, C, C), lambda l, t: (e_(l), 0, 0)),
        pl.BlockSpec((1, 1, C), lambda l, t: (e_(l), 0, 0)),
        pl.BlockSpec((1, CT, C), lambda l, t: (l, mc(t), 0)),
        pl.BlockSpec((1, 1, CT), lambda l, t: (l, 0, mc(t))),
        pl.BlockSpec((1, C, CT), lambda l, t: (l, 0, mc(t))),
        pl.BlockSpec((1, 1, C), lambda l, t: (l, 0, 0)),
        pl.BlockSpec((1, C), lambda l, t: (0, 0)),
        pl.BlockSpec((1, C), lambda l, t: (0, 0)),
        pl.BlockSpec((NS, NS), lambda l, t: (0, 0)),
        pl.BlockSpec((NS, 1), lambda l, t: (0, 0)),
        pl.BlockSpec((1, NP), lambda l, t: (0, 0)),
    ]
    return pl.pallas_call(
        _mega_body,
        grid=(DEPTH, NSTG),
        in_specs=in_specs,
        out_specs=pl.BlockSpec((4, NP, C), lambda l, t: (0, 0, 0)),
        out_shape=jax.ShapeDtypeStruct((4, NP, C), jnp.float32),
        scratch_shapes=[pltpu.VMEM((4, NP, C), jnp.float32),
                        pltpu.VMEM((MB, C), jnp.bfloat16),
                        pltpu.VMEM((MB, 3 * C), jnp.bfloat16),
                        pltpu.VMEM((MB, C), jnp.bfloat16)],
        compiler_params=pltpu.CompilerParams(
            dimension_semantics=("arbitrary", "arbitrary"),
            vmem_limit_bytes=_VMEM_LIMIT),
        name="d4rt_layers",
    )(h, n1w3, n1b3, n2w3, n2b3, lqw, lpw, lpb3, gqw, gqb3, gow, gob3,
      f1w, f1b3, f2w, f2b3, fnw2, fnb2, lb, npad, gb)


def kernel(x, aspect_ratio, conv_w, conv_b, pe_ln_w, pe_ln_b, ar_token,
           t_pos, s_pos, n1_w, n1_b, n2_w, n2_b, loc_qkv_w, loc_proj_w,
           loc_proj_b, glb_in_w, glb_in_b, glb_out_w, glb_out_b,
           fc1_w, fc1_b, fc2_w, fc2_b, fn_w, fn_b):
    B = x.shape[0]
    f32 = jnp.float32

    xp = x.transpose(0, 2, 1, 3, 4).reshape(B, 3, 2, 16, 16, 16, 16)
    patches = xp.transpose(0, 3, 5, 1, 2, 4, 6).reshape(B, NS, PK)
    wpe = conv_w.reshape(C, PK)
    pos = (t_pos[0, 0][None, :] + s_pos[0]).astype(f32)

    row = lambda a: a.reshape(1, -1).astype(f32)
    tokens = _embed(patches, wpe, row(conv_b), row(pe_ln_w), row(pe_ln_b), pos)

    ar_rows = ar_token * (1.0 + aspect_ratio[:, None, None] * 0.1)
    h = jnp.concatenate(
        [tokens, ar_rows.astype(f32), jnp.zeros((B, NP - NT, C), f32)], axis=1)

    lb = jnp.asarray(_LOC_BIAS_NP)
    npad = jnp.asarray(_NPAD_NP)
    gb = jnp.asarray(_GLB_BIAS_NP)

    out = _mega(h,
                n1_w.reshape(DEPTH, 1, C), n1_b.reshape(DEPTH, 1, C),
                n2_w.reshape(DEPTH, 1, C), n2_b.reshape(DEPTH, 1, C),
                loc_qkv_w, loc_proj_w, loc_proj_b.reshape(6, 1, C),
                glb_in_w, glb_in_b.reshape(6, 1, 3 * C),
                glb_out_w, glb_out_b.reshape(6, 1, C),
                fc1_w, fc1_b.reshape(DEPTH, 1, CH),
                fc2_w, fc2_b.reshape(DEPTH, 1, C),
                fn_w.reshape(1, C), fn_b.reshape(1, C),
                lb, npad, gb)
    return out[:, :NT]


# speedup vs baseline: 1.3187x; 1.3187x over previous
import numpy as np
import jax
import jax.numpy as jnp
from jax.experimental import pallas as pl
from jax.experimental.pallas import tpu as pltpu

C = 768
HEADS = 12
DEPTH = 12
HD = C // HEADS
NT = 257
NP = 264
NS = 256
CH = 3351
PK = 1536
SCALE = HD ** -0.5
NEG = -1e30
EPS = 1e-5
CT = 768
MT = -(-CH // CT)
NSTG = 4 + MT
MH = 2 * NP

_VMEM_LIMIT = 50 * 1024 * 1024


def _build_consts():
    g = np.arange(16)
    xx = np.repeat(g, 16)
    yy = np.tile(g, 16)
    win = (np.abs(xx[:, None] - xx[None, :]) <= 3) & (np.abs(yy[:, None] - yy[None, :]) <= 3)
    loc_bias = np.where(win, 0.0, NEG).astype(np.float32)
    nx = np.minimum(xx + 3, 15) - np.maximum(xx - 3, 0) + 1
    ny = np.minimum(yy + 3, 15) - np.maximum(yy - 3, 0) + 1
    npad = (49 - nx * ny).astype(np.float32)
    cm = np.zeros((NP, 3 * 128), np.float32)
    cm[:NS, :NS] = loc_bias
    cm[:NS, NS] = npad
    cm[NS, :NP] = 0.0
    cm[NS, NT:NP] = NEG
    return cm


_CMASK_NP = _build_consts()


def _ln(x, w, b):
    m = jnp.mean(x, axis=-1, keepdims=True)
    xc = x - m
    v = jnp.mean(xc * xc, axis=-1, keepdims=True)
    return xc * jax.lax.rsqrt(v + EPS) * w + b


def _mmt(a, b):
    return jax.lax.dot_general(a, b, (((1,), (1,)), ((), ())),
                               preferred_element_type=jnp.float32)


def _mm(a, b):
    return jax.lax.dot_general(a, b, (((1,), (0,)), ((), ())),
                               preferred_element_type=jnp.float32)


def _gelu(x):
    return 0.5 * x * (1.0 + jax.lax.erf(x * (2.0 ** -0.5)))


def _embed_body(p_ref, w_ref, cb_ref, lnw_ref, lnb_ref, pos_ref, o_ref):
    x = p_ref[...].reshape(2 * NS, PK)
    pe = _mmt(x.astype(jnp.bfloat16), w_ref[...].astype(jnp.bfloat16))
    pe = _ln(pe + cb_ref[...], lnw_ref[...], lnb_ref[...])
    o_ref[...] = pe.reshape(2, NS, C) + pos_ref[...][None]


def _embed(patches, wpe, cb, lnw, lnb, pos):
    full = lambda a: pl.BlockSpec(a.shape, lambda c: (0,) * a.ndim)
    return pl.pallas_call(
        _embed_body,
        grid=(2,),
        in_specs=[pl.BlockSpec((2, NS, PK), lambda c: (c, 0, 0)),
                  full(wpe), full(cb), full(lnw), full(lnb), full(pos)],
        out_specs=pl.BlockSpec((2, NS, C), lambda c: (c, 0, 0)),
        out_shape=jax.ShapeDtypeStruct((4, NS, C), jnp.float32),
        compiler_params=pltpu.CompilerParams(
            dimension_semantics=("parallel",),
            vmem_limit_bytes=_VMEM_LIMIT),
        name="d4rt_embed",
    )(patches, wpe, cb, lnw, lnb, pos)


def _mega_body(h_ref, lnp, lqw, lpw, gqw, gqb, gow, f1w, f1b, f2w, fn_ref,
               cm_ref, o_ref, h_s, ln_s, qkv_s, ao_s):
    l = pl.program_id(1)
    t = pl.program_id(2)
    even = (l % 2) == 0
    bf16 = jnp.bfloat16

    @pl.when((l == 0) & (t == 0))
    def _():
        h_s[...] = h_ref[...]

    @pl.when(t == 0)
    def _():
        ln_s[...] = _ln(h_s[...].reshape(MH, C), lnp[0, 0:1], lnp[0, 1:2]
                        ).astype(bf16)

    @pl.when(t < 3)
    def _():
        col = pl.ds(t * C, C)

        @pl.when(even)
        def _():
            qkv_s[:, col] = _mmt(ln_s[...], lqw[0].astype(bf16)).astype(bf16)

        @pl.when(~even)
        def _():
            qkv_s[:, col] = (_mmt(ln_s[...], gqw[0].astype(bf16))
                             + gqb[0]).astype(bf16)

    @pl.when(t == 3)
    def _():
        @pl.when(even)
        def _():
            lb = cm_ref[:NS, :NS]
            npad = cm_ref[:NS, NS:NS + 1]
            for b in range(2):
                r0 = b * NP
                for hh in range(HEADS):
                    c0 = hh * HD
                    q = qkv_s[r0:r0 + NS, c0:c0 + HD]
                    k = qkv_s[r0:r0 + NS, C + c0:C + c0 + HD]
                    v = qkv_s[r0:r0 + NS, 2 * C + c0:2 * C + c0 + HD]
                    s = _mmt(q, k) * SCALE + lb
                    m = jnp.maximum(jnp.max(s, axis=-1, keepdims=True), 0.0)
                    e = jnp.exp(s - m)
                    den = (jnp.sum(e, axis=-1, keepdims=True)
                           + npad * jnp.exp(-m))
                    ao_s[r0:r0 + NS, c0:c0 + HD] = _mm(
                        (e / den).astype(bf16), v).astype(bf16)
            out = _mmt(ao_s[...], lpw[0].astype(bf16)) + lnp[0, 4:5]
            o2 = out.reshape(2, NP, C)
            h_s[:, :NS, :] = h_s[:, :NS, :] + o2[:, :NS, :]

        @pl.when(~even)
        def _():
            gb = cm_ref[NS:NS + 1, :NP]
            for b in range(2):
                r0 = b * NP
                for hh in range(HEADS):
                    c0 = hh * HD
                    q = qkv_s[r0:r0 + NP, c0:c0 + HD]
                    k = qkv_s[r0:r0 + NP, C + c0:C + c0 + HD]
                    v = qkv_s[r0:r0 + NP, 2 * C + c0:2 * C + c0 + HD]
                    s = _mmt(q, k) * SCALE + gb
                    m = jnp.max(s, axis=-1, keepdims=True)
                    e = jnp.exp(s - m)
                    p = e / jnp.sum(e, axis=-1, keepdims=True)
                    ao_s[r0:r0 + NP, c0:c0 + HD] = _mm(
                        p.astype(bf16), v).astype(bf16)
            out = _mmt(ao_s[...], gow[0].astype(bf16)) + lnp[0, 4:5]
            h_s[...] = h_s[...] + out.reshape(2, NP, C)

        ln_s[...] = _ln(h_s[...].reshape(MH, C), lnp[0, 2:3], lnp[0, 3:4]
                        ).astype(bf16)

    @pl.when(t >= 4)
    def _():
        hcol = _mmt(ln_s[...], f1w[0].astype(bf16)) + f1b[0]
        lane = (t - 4) * CT + jax.lax.broadcasted_iota(jnp.int32, (1, CT), 1)
        valid = lane < CH
        hcol = jnp.where(valid, _gelu(hcol), 0.0).astype(bf16)
        f2wv = jnp.where(valid, f2w[0].astype(bf16), jnp.bfloat16(0.0))
        part = _mmt(hcol, f2wv).reshape(2, NP, C)

        @pl.when(t < NSTG - 1)
        def _():
            h_s[...] = h_s[...] + part

        @pl.when(t == NSTG - 1)
        def _():
            fin = h_s[...].reshape(MH, C) + part.reshape(MH, C) + lnp[0, 5:6]
            h_s[...] = fin.reshape(2, NP, C)

            @pl.when(l == DEPTH - 1)
            def _():
                o_ref[...] = _ln(fin, fn_ref[0:1], fn_ref[1:2]
                                 ).reshape(2, NP, C)


def _mega(h, lnp, lqw, lpw, gqw, gqb3, gow, f1w, f1b3, f2w, fn, cm):
    qc = lambda t: jnp.minimum(t, 2)
    mc = lambda t: jnp.clip(t - 4, 0, MT - 1)
    e_ = lambda l: l // 2
    even_ = lambda l: (l % 2) == 0
    lq_c = lambda l, t: jnp.where(even_(l), qc(t), 2)
    gq_c = lambda l, t: jnp.where(even_(l), 2, qc(t))

    in_specs = [
        pl.BlockSpec((2, NP, C), lambda c, l, t: (c, 0, 0)),
        pl.BlockSpec((1, 6, C), lambda c, l, t: (l, 0, 0)),
        pl.BlockSpec((1, C, C), lambda c, l, t: (e_(l), lq_c(l, t), 0)),
        pl.BlockSpec((1, C, C), lambda c, l, t: (e_(l), 0, 0)),
        pl.BlockSpec((1, C, C), lambda c, l, t: (e_(l), gq_c(l, t), 0)),
        pl.BlockSpec((1, 1, C), lambda c, l, t: (e_(l), 0, gq_c(l, t))),
        pl.BlockSpec((1, C, C), lambda c, l, t: (e_(l), 0, 0)),
        pl.BlockSpec((1, CT, C), lambda c, l, t: (l, mc(t), 0)),
        pl.BlockSpec((1, 1, CT), lambda c, l, t: (l, 0, mc(t))),
        pl.BlockSpec((1, C, CT), lambda c, l, t: (l, 0, mc(t))),
        pl.BlockSpec((2, C), lambda c, l, t: (0, 0)),
        pl.BlockSpec((NP, 3 * 128), lambda c, l, t: (0, 0)),
    ]
    return pl.pallas_call(
        _mega_body,
        grid=(2, DEPTH, NSTG),
        in_specs=in_specs,
        out_specs=pl.BlockSpec((2, NP, C), lambda c, l, t: (c, 0, 0)),
        out_shape=jax.ShapeDtypeStruct((4, NP, C), jnp.float32),
        scratch_shapes=[pltpu.VMEM((2, NP, C), jnp.float32),
                        pltpu.VMEM((MH, C), jnp.bfloat16),
                        pltpu.VMEM((MH, 3 * C), jnp.bfloat16),
                        pltpu.VMEM((MH, C), jnp.bfloat16)],
        compiler_params=pltpu.CompilerParams(
            dimension_semantics=("parallel", "arbitrary", "arbitrary"),
            vmem_limit_bytes=_VMEM_LIMIT),
        name="d4rt_layers",
    )(h, lnp, lqw, lpw, gqw, gqb3, gow, f1w, f1b3, f2w, fn, cm)


def kernel(x, aspect_ratio, conv_w, conv_b, pe_ln_w, pe_ln_b, ar_token,
           t_pos, s_pos, n1_w, n1_b, n2_w, n2_b, loc_qkv_w, loc_proj_w,
           loc_proj_b, glb_in_w, glb_in_b, glb_out_w, glb_out_b,
           fc1_w, fc1_b, fc2_w, fc2_b, fn_w, fn_b):
    B = x.shape[0]
    f32 = jnp.float32

    xp = x.transpose(0, 2, 1, 3, 4).reshape(B, 3, 2, 16, 16, 16, 16)
    patches = xp.transpose(0, 3, 5, 1, 2, 4, 6).reshape(B, NS, PK)
    wpe = conv_w.reshape(C, PK)
    pos = (t_pos[0, 0][None, :] + s_pos[0]).astype(f32)

    row = lambda a: a.reshape(1, -1).astype(f32)
    tokens = _embed(patches, wpe, row(conv_b), row(pe_ln_w), row(pe_ln_b), pos)

    ar_rows = ar_token * (1.0 + aspect_ratio[:, None, None] * 0.1)
    h = jnp.concatenate(
        [tokens, ar_rows.astype(f32), jnp.zeros((B, NP - NT, C), f32)], axis=1)

    projb = jnp.stack([loc_proj_b, glb_out_b], axis=1).reshape(DEPTH, C)
    lnp = jnp.stack([n1_w, n1_b, n2_w, n2_b, projb, fc2_b], axis=1)
    fn = jnp.stack([fn_w, fn_b])
    cm = jnp.asarray(_CMASK_NP)

    out = _mega(h, lnp, loc_qkv_w, loc_proj_w, glb_in_w,
                glb_in_b.reshape(6, 1, 3 * C), glb_out_w,
                fc1_w, fc1_b.reshape(DEPTH, 1, CH), fc2_w, fn, cm)
    return out[:, :NT]


# speedup vs baseline: 1.7106x; 1.2972x over previous
import numpy as np
import jax
import jax.numpy as jnp
from jax.experimental import pallas as pl
from jax.experimental.pallas import tpu as pltpu

C = 768
HEADS = 12
DEPTH = 12
HD = C // HEADS
NT = 257
NP = 264
NS = 256
CH = 3351
PK = 1536
SCALE = HD ** -0.5
NEG = -1e30
EPS = 1e-5
CT = 768
MT = -(-CH // CT)
NSTG = 4 + MT
MH = 2 * NP

_VMEM_LIMIT = 50 * 1024 * 1024


def _build_consts():
    g = np.arange(16)
    xx = np.repeat(g, 16)
    yy = np.tile(g, 16)
    win = (np.abs(xx[:, None] - xx[None, :]) <= 3) & (np.abs(yy[:, None] - yy[None, :]) <= 3)
    loc_bias = np.where(win, 0.0, NEG).astype(np.float32)
    nx = np.minimum(xx + 3, 15) - np.maximum(xx - 3, 0) + 1
    ny = np.minimum(yy + 3, 15) - np.maximum(yy - 3, 0) + 1
    npad = (49 - nx * ny).astype(np.float32)
    cm = np.zeros((NP, 3 * 128), np.float32)
    cm[:NS, :NS] = loc_bias
    cm[:NS, NS] = npad
    cm[NS, :NP] = 0.0
    cm[NS, NT:NP] = NEG
    return cm


_CMASK_NP = _build_consts()


def _ln(x, w, b):
    m = jnp.mean(x, axis=-1, keepdims=True)
    xc = x - m
    v = jnp.mean(xc * xc, axis=-1, keepdims=True)
    return xc * jax.lax.rsqrt(v + EPS) * w + b


def _mmt(a, b):
    return jax.lax.dot_general(a, b, (((1,), (1,)), ((), ())),
                               preferred_element_type=jnp.float32)


def _mm(a, b):
    return jax.lax.dot_general(a, b, (((1,), (0,)), ((), ())),
                               preferred_element_type=jnp.float32)


def _gelu(x):
    return 0.5 * x * (1.0 + jax.lax.erf(x * (2.0 ** -0.5)))


def _embed_body(p_ref, w_ref, cb_ref, lnw_ref, lnb_ref, pos_ref, o_ref):
    x = p_ref[...].reshape(2 * NS, PK)
    pe = _mmt(x.astype(jnp.bfloat16), w_ref[...].astype(jnp.bfloat16))
    pe = _ln(pe + cb_ref[...], lnw_ref[...], lnb_ref[...])
    o_ref[...] = pe.reshape(2, NS, C) + pos_ref[...][None]


def _embed(patches, wpe, cb, lnw, lnb, pos):
    full = lambda a: pl.BlockSpec(a.shape, lambda c: (0,) * a.ndim)
    return pl.pallas_call(
        _embed_body,
        grid=(2,),
        in_specs=[pl.BlockSpec((2, NS, PK), lambda c: (c, 0, 0)),
                  full(wpe), full(cb), full(lnw), full(lnb), full(pos)],
        out_specs=pl.BlockSpec((2, NS, C), lambda c: (c, 0, 0)),
        out_shape=jax.ShapeDtypeStruct((4, NS, C), jnp.float32),
        compiler_params=pltpu.CompilerParams(
            dimension_semantics=("parallel",),
            vmem_limit_bytes=_VMEM_LIMIT),
        name="d4rt_embed",
    )(patches, wpe, cb, lnw, lnb, pos)


def _mega_body(h_ref, lnp, lqw, lpw, gqw, gqb, gow, f1w, f1b, f2w, fn_ref,
               cm_ref, o_ref, h_s, ln_s, qkv_s, ao_s):
    l = pl.program_id(1)
    t = pl.program_id(2)
    even = (l % 2) == 0
    bf16 = jnp.bfloat16

    @pl.when((l == 0) & (t == 0))
    def _():
        h_s[...] = h_ref[...]

    @pl.when(t == 0)
    def _():
        ln_s[...] = _ln(h_s[...].reshape(MH, C), lnp[0, 0:1], lnp[0, 1:2]
                        ).astype(bf16)

    @pl.when(t < 3)
    def _():
        col = pl.ds(t * C, C)

        @pl.when(even)
        def _():
            qkv_s[:, col] = _mmt(ln_s[...], lqw[0].astype(bf16)).astype(bf16)

        @pl.when(~even)
        def _():
            qkv_s[:, col] = (_mmt(ln_s[...], gqw[0].astype(bf16))
                             + gqb[0]).astype(bf16)

    @pl.when(t == 3)
    def _():
        @pl.when(even)
        def _():
            lb = cm_ref[:NS, :NS]
            npad = cm_ref[:NS, NS:NS + 1]
            ones = jnp.ones((NS, 1), bf16)
            for b in range(2):
                r0 = b * NP
                for hh in range(HEADS):
                    c0 = hh * HD
                    q = qkv_s[r0:r0 + NS, c0:c0 + HD]
                    k = qkv_s[r0:r0 + NS, C + c0:C + c0 + HD]
                    v = qkv_s[r0:r0 + NS, 2 * C + c0:2 * C + c0 + HD]
                    e = jnp.exp(_mmt(q, k) * SCALE + lb)
                    ve = jnp.concatenate([v, ones], axis=1)
                    oe = _mm(e.astype(bf16), ve)
                    den = oe[:, HD:HD + 1] + npad
                    ao_s[r0:r0 + NS, c0:c0 + HD] = (
                        oe[:, :HD] / den).astype(bf16)
            out = _mmt(ao_s[...], lpw[0].astype(bf16)) + lnp[0, 4:5]
            o2 = out.reshape(2, NP, C)
            h_s[:, :NS, :] = h_s[:, :NS, :] + o2[:, :NS, :]

        @pl.when(~even)
        def _():
            gb = cm_ref[NS:NS + 1, :NP]
            ones = jnp.ones((NP, 1), bf16)
            for b in range(2):
                r0 = b * NP
                for hh in range(HEADS):
                    c0 = hh * HD
                    q = qkv_s[r0:r0 + NP, c0:c0 + HD]
                    k = qkv_s[r0:r0 + NP, C + c0:C + c0 + HD]
                    v = qkv_s[r0:r0 + NP, 2 * C + c0:2 * C + c0 + HD]
                    e = jnp.exp(_mmt(q, k) * SCALE + gb)
                    ve = jnp.concatenate([v, ones], axis=1)
                    oe = _mm(e.astype(bf16), ve)
                    ao_s[r0:r0 + NP, c0:c0 + HD] = (
                        oe[:, :HD] / oe[:, HD:HD + 1]).astype(bf16)
            out = _mmt(ao_s[...], gow[0].astype(bf16)) + lnp[0, 4:5]
            h_s[...] = h_s[...] + out.reshape(2, NP, C)

        ln_s[...] = _ln(h_s[...].reshape(MH, C), lnp[0, 2:3], lnp[0, 3:4]
                        ).astype(bf16)

    @pl.when(t >= 4)
    def _():
        hcol = _mmt(ln_s[...], f1w[0].astype(bf16)) + f1b[0]
        lane = (t - 4) * CT + jax.lax.broadcasted_iota(jnp.int32, (1, CT), 1)
        valid = lane < CH
        hcol = jnp.where(valid, _gelu(hcol), 0.0).astype(bf16)
        f2wv = jnp.where(valid, f2w[0].astype(bf16), jnp.bfloat16(0.0))
        part = _mmt(hcol, f2wv).reshape(2, NP, C)

        @pl.when(t < NSTG - 1)
        def _():
            h_s[...] = h_s[...] + part

        @pl.when(t == NSTG - 1)
        def _():
            fin = h_s[...].reshape(MH, C) + part.reshape(MH, C) + lnp[0, 5:6]
            h_s[...] = fin.reshape(2, NP, C)

            @pl.when(l == DEPTH - 1)
            def _():
                o_ref[...] = _ln(fin, fn_ref[0:1], fn_ref[1:2]
                                 ).reshape(2, NP, C)


def _mega(h, lnp, lqw, lpw, gqw, gqb3, gow, f1w, f1b3, f2w, fn, cm):
    qc = lambda t: jnp.minimum(t, 2)
    mc = lambda t: jnp.clip(t - 4, 0, MT - 1)
    e_ = lambda l: l // 2
    even_ = lambda l: (l % 2) == 0
    lq_c = lambda l, t: jnp.where(even_(l), qc(t), 2)
    gq_c = lambda l, t: jnp.where(even_(l), 2, qc(t))

    in_specs = [
        pl.BlockSpec((2, NP, C), lambda c, l, t: (c, 0, 0)),
        pl.BlockSpec((1, 6, C), lambda c, l, t: (l, 0, 0)),
        pl.BlockSpec((1, C, C), lambda c, l, t: (e_(l), lq_c(l, t), 0)),
        pl.BlockSpec((1, C, C), lambda c, l, t: (e_(l), 0, 0)),
        pl.BlockSpec((1, C, C), lambda c, l, t: (e_(l), gq_c(l, t), 0)),
        pl.BlockSpec((1, 1, C), lambda c, l, t: (e_(l), 0, gq_c(l, t))),
        pl.BlockSpec((1, C, C), lambda c, l, t: (e_(l), 0, 0)),
        pl.BlockSpec((1, CT, C), lambda c, l, t: (l, mc(t), 0)),
        pl.BlockSpec((1, 1, CT), lambda c, l, t: (l, 0, mc(t))),
        pl.BlockSpec((1, C, CT), lambda c, l, t: (l, 0, mc(t))),
        pl.BlockSpec((2, C), lambda c, l, t: (0, 0)),
        pl.BlockSpec((NP, 3 * 128), lambda c, l, t: (0, 0)),
    ]
    return pl.pallas_call(
        _mega_body,
        grid=(2, DEPTH, NSTG),
        in_specs=in_specs,
        out_specs=pl.BlockSpec((2, NP, C), lambda c, l, t: (c, 0, 0)),
        out_shape=jax.ShapeDtypeStruct((4, NP, C), jnp.float32),
        scratch_shapes=[pltpu.VMEM((2, NP, C), jnp.float32),
                        pltpu.VMEM((MH, C), jnp.bfloat16),
                        pltpu.VMEM((MH, 3 * C), jnp.bfloat16),
                        pltpu.VMEM((MH, C), jnp.bfloat16)],
        compiler_params=pltpu.CompilerParams(
            dimension_semantics=("parallel", "arbitrary", "arbitrary"),
            vmem_limit_bytes=_VMEM_LIMIT),
        name="d4rt_layers",
    )(h, lnp, lqw, lpw, gqw, gqb3, gow, f1w, f1b3, f2w, fn, cm)


def kernel(x, aspect_ratio, conv_w, conv_b, pe_ln_w, pe_ln_b, ar_token,
           t_pos, s_pos, n1_w, n1_b, n2_w, n2_b, loc_qkv_w, loc_proj_w,
           loc_proj_b, glb_in_w, glb_in_b, glb_out_w, glb_out_b,
           fc1_w, fc1_b, fc2_w, fc2_b, fn_w, fn_b):
    B = x.shape[0]
    f32 = jnp.float32

    xp = x.transpose(0, 2, 1, 3, 4).reshape(B, 3, 2, 16, 16, 16, 16)
    patches = xp.transpose(0, 3, 5, 1, 2, 4, 6).reshape(B, NS, PK)
    wpe = conv_w.reshape(C, PK)
    pos = (t_pos[0, 0][None, :] + s_pos[0]).astype(f32)

    row = lambda a: a.reshape(1, -1).astype(f32)
    tokens = _embed(patches, wpe, row(conv_b), row(pe_ln_w), row(pe_ln_b), pos)

    ar_rows = ar_token * (1.0 + aspect_ratio[:, None, None] * 0.1)
    h = jnp.concatenate(
        [tokens, ar_rows.astype(f32), jnp.zeros((B, NP - NT, C), f32)], axis=1)

    projb = jnp.stack([loc_proj_b, glb_out_b], axis=1).reshape(DEPTH, C)
    lnp = jnp.stack([n1_w, n1_b, n2_w, n2_b, projb, fc2_b], axis=1)
    fn = jnp.stack([fn_w, fn_b])
    cm = jnp.asarray(_CMASK_NP)

    out = _mega(h, lnp, loc_qkv_w, loc_proj_w, glb_in_w,
                glb_in_b.reshape(6, 1, 3 * C), glb_out_w,
                fc1_w, fc1_b.reshape(DEPTH, 1, CH), fc2_w, fn, cm)
    return out[:, :NT]


# speedup vs baseline: 1.8420x; 1.0768x over previous
import numpy as np
import jax
import jax.numpy as jnp
from jax.experimental import pallas as pl
from jax.experimental.pallas import tpu as pltpu

C = 768
HEADS = 12
DEPTH = 12
HD = C // HEADS
NT = 257
NP = 264
NS = 256
CH = 3351
PK = 1536
SCALE = HD ** -0.5
NEG = -1e30
EPS = 1e-5
CT = 1152
MT = -(-CH // CT)
NSTG = 4 + MT
MH = 2 * NP

_VMEM_LIMIT = 50 * 1024 * 1024


def _build_consts():
    g = np.arange(16)
    xx = np.repeat(g, 16)
    yy = np.tile(g, 16)
    win = (np.abs(xx[:, None] - xx[None, :]) <= 3) & (np.abs(yy[:, None] - yy[None, :]) <= 3)
    loc_bias = np.where(win, 0.0, NEG).astype(np.float32)
    nx = np.minimum(xx + 3, 15) - np.maximum(xx - 3, 0) + 1
    ny = np.minimum(yy + 3, 15) - np.maximum(yy - 3, 0) + 1
    npad = (49 - nx * ny).astype(np.float32)
    cm = np.zeros((NP, 3 * 128), np.float32)
    cm[:NS, :NS] = loc_bias
    cm[:NS, NS] = npad
    cm[NS, :NP] = 0.0
    cm[NS, NT:NP] = NEG
    return cm


_CMASK_NP = _build_consts()


def _ln(x, w, b):
    m = jnp.mean(x, axis=-1, keepdims=True)
    xc = x - m
    v = jnp.mean(xc * xc, axis=-1, keepdims=True)
    return xc * jax.lax.rsqrt(v + EPS) * w + b


def _mmt(a, b):
    return jax.lax.dot_general(a, b, (((1,), (1,)), ((), ())),
                               preferred_element_type=jnp.float32)


def _mm(a, b):
    return jax.lax.dot_general(a, b, (((1,), (0,)), ((), ())),
                               preferred_element_type=jnp.float32)


def _gelu(x):
    return 0.5 * x * (1.0 + jax.lax.erf(x * (2.0 ** -0.5)))


def _embed_body(p_ref, w_ref, cb_ref, lnw_ref, lnb_ref, pos_ref, o_ref):
    x = p_ref[...].reshape(2 * NS, PK)
    pe = _mmt(x.astype(jnp.bfloat16), w_ref[...].astype(jnp.bfloat16))
    pe = _ln(pe + cb_ref[...], lnw_ref[...], lnb_ref[...])
    o_ref[...] = pe.reshape(2, NS, C) + pos_ref[...][None]


def _embed(patches, wpe, cb, lnw, lnb, pos):
    full = lambda a: pl.BlockSpec(a.shape, lambda c: (0,) * a.ndim)
    return pl.pallas_call(
        _embed_body,
        grid=(2,),
        in_specs=[pl.BlockSpec((2, NS, PK), lambda c: (c, 0, 0)),
                  full(wpe), full(cb), full(lnw), full(lnb), full(pos)],
        out_specs=pl.BlockSpec((2, NS, C), lambda c: (c, 0, 0)),
        out_shape=jax.ShapeDtypeStruct((4, NS, C), jnp.float32),
        compiler_params=pltpu.CompilerParams(
            dimension_semantics=("parallel",),
            vmem_limit_bytes=_VMEM_LIMIT),
        name="d4rt_embed",
    )(patches, wpe, cb, lnw, lnb, pos)


def _mega_body(h_ref, lnp, lqw, lpw, gqw, gqb, gow, f1w, f1b, f2w, fn_ref,
               cm_ref, o_ref, h_s, ln_s, qkv_s, ao_s):
    l = pl.program_id(1)
    t = pl.program_id(2)
    even = (l % 2) == 0
    bf16 = jnp.bfloat16

    @pl.when((l == 0) & (t == 0))
    def _():
        h_s[...] = h_ref[...]

    @pl.when(t == 0)
    def _():
        ln_s[...] = _ln(h_s[...].reshape(MH, C), lnp[0, 0:1], lnp[0, 1:2]
                        ).astype(bf16)

    @pl.when(t < 3)
    def _():
        col = pl.ds(t * C, C)

        @pl.when(even)
        def _():
            qkv_s[:, col] = _mmt(ln_s[...], lqw[0].astype(bf16)).astype(bf16)

        @pl.when(~even)
        def _():
            qkv_s[:, col] = (_mmt(ln_s[...], gqw[0].astype(bf16))
                             + gqb[0]).astype(bf16)

    @pl.when(t == 3)
    def _():
        @pl.when(even)
        def _():
            lb = cm_ref[:NS, :NS]
            npad = cm_ref[:NS, NS:NS + 1]
            ones = jnp.ones((NS, 1), bf16)
            for b in range(2):
                r0 = b * NP
                for hh in range(HEADS):
                    c0 = hh * HD
                    q = qkv_s[r0:r0 + NS, c0:c0 + HD]
                    k = qkv_s[r0:r0 + NS, C + c0:C + c0 + HD]
                    v = qkv_s[r0:r0 + NS, 2 * C + c0:2 * C + c0 + HD]
                    e = jnp.exp(_mmt(q, k) * SCALE + lb)
                    ve = jnp.concatenate([v, ones], axis=1)
                    oe = _mm(e.astype(bf16), ve)
                    den = oe[:, HD:HD + 1] + npad
                    ao_s[r0:r0 + NS, c0:c0 + HD] = (
                        oe[:, :HD] / den).astype(bf16)
            out = _mmt(ao_s[...], lpw[0].astype(bf16)) + lnp[0, 4:5]
            o2 = out.reshape(2, NP, C)
            h_s[:, :NS, :] = h_s[:, :NS, :] + o2[:, :NS, :]

        @pl.when(~even)
        def _():
            gb = cm_ref[NS:NS + 1, :NP]
            ones = jnp.ones((NP, 1), bf16)
            for b in range(2):
                r0 = b * NP
                for hh in range(HEADS):
                    c0 = hh * HD
                    q = qkv_s[r0:r0 + NP, c0:c0 + HD]
                    k = qkv_s[r0:r0 + NP, C + c0:C + c0 + HD]
                    v = qkv_s[r0:r0 + NP, 2 * C + c0:2 * C + c0 + HD]
                    e = jnp.exp(_mmt(q, k) * SCALE + gb)
                    ve = jnp.concatenate([v, ones], axis=1)
                    oe = _mm(e.astype(bf16), ve)
                    ao_s[r0:r0 + NP, c0:c0 + HD] = (
                        oe[:, :HD] / oe[:, HD:HD + 1]).astype(bf16)
            out = _mmt(ao_s[...], gow[0].astype(bf16)) + lnp[0, 4:5]
            h_s[...] = h_s[...] + out.reshape(2, NP, C)

        ln_s[...] = _ln(h_s[...].reshape(MH, C), lnp[0, 2:3], lnp[0, 3:4]
                        ).astype(bf16)

    @pl.when(t >= 4)
    def _():
        hcol = _mmt(ln_s[...], f1w[0].astype(bf16)) + f1b[0]
        lane = (t - 4) * CT + jax.lax.broadcasted_iota(jnp.int32, (1, CT), 1)
        valid = lane < CH
        hcol = jnp.where(valid, _gelu(hcol), 0.0).astype(bf16)
        f2wv = jnp.where(valid, f2w[0].astype(bf16), jnp.bfloat16(0.0))
        part = _mmt(hcol, f2wv).reshape(2, NP, C)

        @pl.when(t < NSTG - 1)
        def _():
            h_s[...] = h_s[...] + part

        @pl.when(t == NSTG - 1)
        def _():
            fin = h_s[...].reshape(MH, C) + part.reshape(MH, C) + lnp[0, 5:6]
            h_s[...] = fin.reshape(2, NP, C)

            @pl.when(l == DEPTH - 1)
            def _():
                o_ref[...] = _ln(fin, fn_ref[0:1], fn_ref[1:2]
                                 ).reshape(2, NP, C)


def _mega(h, lnp, lqw, lpw, gqw, gqb3, gow, f1w, f1b3, f2w, fn, cm):
    qc = lambda t: jnp.minimum(t, 2)
    mc = lambda t: jnp.clip(t - 4, 0, MT - 1)
    e_ = lambda l: l // 2
    even_ = lambda l: (l % 2) == 0
    lq_c = lambda l, t: jnp.where(even_(l), qc(t), 2)
    gq_c = lambda l, t: jnp.where(even_(l), 2, qc(t))

    in_specs = [
        pl.BlockSpec((2, NP, C), lambda c, l, t: (c, 0, 0)),
        pl.BlockSpec((1, 6, C), lambda c, l, t: (l, 0, 0)),
        pl.BlockSpec((1, C, C), lambda c, l, t: (e_(l), lq_c(l, t), 0)),
        pl.BlockSpec((1, C, C), lambda c, l, t: (e_(l), 0, 0)),
        pl.BlockSpec((1, C, C), lambda c, l, t: (e_(l), gq_c(l, t), 0)),
        pl.BlockSpec((1, 1, C), lambda c, l, t: (e_(l), 0, gq_c(l, t))),
        pl.BlockSpec((1, C, C), lambda c, l, t: (e_(l), 0, 0)),
        pl.BlockSpec((1, CT, C), lambda c, l, t: (l, mc(t), 0)),
        pl.BlockSpec((1, 1, CT), lambda c, l, t: (l, 0, mc(t))),
        pl.BlockSpec((1, C, CT), lambda c, l, t: (l, 0, mc(t))),
        pl.BlockSpec((2, C), lambda c, l, t: (0, 0)),
        pl.BlockSpec((NP, 3 * 128), lambda c, l, t: (0, 0)),
    ]
    return pl.pallas_call(
        _mega_body,
        grid=(2, DEPTH, NSTG),
        in_specs=in_specs,
        out_specs=pl.BlockSpec((2, NP, C), lambda c, l, t: (c, 0, 0)),
        out_shape=jax.ShapeDtypeStruct((4, NP, C), jnp.float32),
        scratch_shapes=[pltpu.VMEM((2, NP, C), jnp.float32),
                        pltpu.VMEM((MH, C), jnp.bfloat16),
                        pltpu.VMEM((MH, 3 * C), jnp.bfloat16),
                        pltpu.VMEM((MH, C), jnp.bfloat16)],
        compiler_params=pltpu.CompilerParams(
            dimension_semantics=("parallel", "arbitrary", "arbitrary"),
            vmem_limit_bytes=_VMEM_LIMIT),
        name="d4rt_layers",
    )(h, lnp, lqw, lpw, gqw, gqb3, gow, f1w, f1b3, f2w, fn, cm)


def kernel(x, aspect_ratio, conv_w, conv_b, pe_ln_w, pe_ln_b, ar_token,
           t_pos, s_pos, n1_w, n1_b, n2_w, n2_b, loc_qkv_w, loc_proj_w,
           loc_proj_b, glb_in_w, glb_in_b, glb_out_w, glb_out_b,
           fc1_w, fc1_b, fc2_w, fc2_b, fn_w, fn_b):
    B = x.shape[0]
    f32 = jnp.float32

    xp = x.transpose(0, 2, 1, 3, 4).reshape(B, 3, 2, 16, 16, 16, 16)
    patches = xp.transpose(0, 3, 5, 1, 2, 4, 6).reshape(B, NS, PK)
    wpe = conv_w.reshape(C, PK)
    pos = (t_pos[0, 0][None, :] + s_pos[0]).astype(f32)

    row = lambda a: a.reshape(1, -1).astype(f32)
    tokens = _embed(patches, wpe, row(conv_b), row(pe_ln_w), row(pe_ln_b), pos)

    ar_rows = ar_token * (1.0 + aspect_ratio[:, None, None] * 0.1)
    h = jnp.concatenate(
        [tokens, ar_rows.astype(f32), jnp.zeros((B, NP - NT, C), f32)], axis=1)

    projb = jnp.stack([loc_proj_b, glb_out_b], axis=1).reshape(DEPTH, C)
    lnp = jnp.stack([n1_w, n1_b, n2_w, n2_b, projb, fc2_b], axis=1)
    fn = jnp.stack([fn_w, fn_b])
    cm = jnp.asarray(_CMASK_NP)

    out = _mega(h, lnp, loc_qkv_w, loc_proj_w, glb_in_w,
                glb_in_b.reshape(6, 1, 3 * C), glb_out_w,
                fc1_w, fc1_b.reshape(DEPTH, 1, CH), fc2_w, fn, cm)
    return out[:, :NT]


# speedup vs baseline: 1.8467x; 1.0026x over previous
import numpy as np
import jax
import jax.numpy as jnp
from jax.experimental import pallas as pl
from jax.experimental.pallas import tpu as pltpu

C = 768
HEADS = 12
DEPTH = 12
HD = C // HEADS
NT = 257
NP = 264
NS = 256
CH = 3351
PK = 1536
SCALE = HD ** -0.5
NEG = -1e30
EPS = 1e-5
CT = 1152
MT = -(-CH // CT)
QC = 1152
NSTG = 3 + MT
MH = 2 * NP

_VMEM_LIMIT = 52 * 1024 * 1024


def _build_consts():
    g = np.arange(16)
    xx = np.repeat(g, 16)
    yy = np.tile(g, 16)
    win = (np.abs(xx[:, None] - xx[None, :]) <= 3) & (np.abs(yy[:, None] - yy[None, :]) <= 3)
    loc_bias = np.where(win, 0.0, NEG).astype(np.float32)
    nx = np.minimum(xx + 3, 15) - np.maximum(xx - 3, 0) + 1
    ny = np.minimum(yy + 3, 15) - np.maximum(yy - 3, 0) + 1
    npad = (49 - nx * ny).astype(np.float32)
    cm = np.zeros((NP, 3 * 128), np.float32)
    cm[:NS, :NS] = loc_bias
    cm[:NS, NS] = npad
    cm[NS, :NP] = 0.0
    cm[NS, NT:NP] = NEG
    return cm


_CMASK_NP = _build_consts()


def _ln(x, w, b):
    m = jnp.mean(x, axis=-1, keepdims=True)
    xc = x - m
    v = jnp.mean(xc * xc, axis=-1, keepdims=True)
    return xc * jax.lax.rsqrt(v + EPS) * w + b


def _mmt(a, b):
    return jax.lax.dot_general(a, b, (((1,), (1,)), ((), ())),
                               preferred_element_type=jnp.float32)


def _mm(a, b):
    return jax.lax.dot_general(a, b, (((1,), (0,)), ((), ())),
                               preferred_element_type=jnp.float32)


def _gelu(x):
    return 0.5 * x * (1.0 + jax.lax.erf(x * (2.0 ** -0.5)))


def _embed_body(p_ref, w_ref, cb_ref, lnw_ref, lnb_ref, pos_ref, o_ref):
    x = p_ref[...].reshape(2 * NS, PK)
    pe = _mmt(x.astype(jnp.bfloat16), w_ref[...].astype(jnp.bfloat16))
    pe = _ln(pe + cb_ref[...], lnw_ref[...], lnb_ref[...])
    o_ref[...] = pe.reshape(2, NS, C) + pos_ref[...][None]


def _embed(patches, wpe, cb, lnw, lnb, pos):
    full = lambda a: pl.BlockSpec(a.shape, lambda c: (0,) * a.ndim)
    return pl.pallas_call(
        _embed_body,
        grid=(2,),
        in_specs=[pl.BlockSpec((2, NS, PK), lambda c: (c, 0, 0)),
                  full(wpe), full(cb), full(lnw), full(lnb), full(pos)],
        out_specs=pl.BlockSpec((2, NS, C), lambda c: (c, 0, 0)),
        out_shape=jax.ShapeDtypeStruct((4, NS, C), jnp.float32),
        compiler_params=pltpu.CompilerParams(
            dimension_semantics=("parallel",),
            vmem_limit_bytes=_VMEM_LIMIT),
        name="d4rt_embed",
    )(patches, wpe, cb, lnw, lnb, pos)


def _mega_body(h_ref, lnp, lqw, lpw, gqw, gqb, gow, f1w, f1b, f2w, fn_ref,
               cm_ref, o_ref, h_s, ln_s, qkv_s, ao_s):
    l = pl.program_id(1)
    t = pl.program_id(2)
    even = (l % 2) == 0
    bf16 = jnp.bfloat16

    @pl.when((l == 0) & (t == 0))
    def _():
        h_s[...] = h_ref[...]

    @pl.when(t == 0)
    def _():
        ln_s[...] = _ln(h_s[...].reshape(MH, C), lnp[0, 0:1], lnp[0, 1:2]
                        ).astype(bf16)

    @pl.when(t < 2)
    def _():
        col = pl.ds(t * QC, QC)

        @pl.when(even)
        def _():
            qkv_s[:, col] = _mmt(ln_s[...], lqw[0].astype(bf16)).astype(bf16)

        @pl.when(~even)
        def _():
            qkv_s[:, col] = (_mmt(ln_s[...], gqw[0].astype(bf16))
                             + gqb[0]).astype(bf16)

    @pl.when(t == 2)
    def _():
        @pl.when(even)
        def _():
            lb = cm_ref[:NS, :NS]
            npad = cm_ref[:NS, NS:NS + 1]
            ones = jnp.ones((NS, 1), bf16)
            for b in range(2):
                r0 = b * NP
                for hh in range(HEADS):
                    c0 = hh * HD
                    q = qkv_s[r0:r0 + NS, c0:c0 + HD]
                    k = qkv_s[r0:r0 + NS, C + c0:C + c0 + HD]
                    v = qkv_s[r0:r0 + NS, 2 * C + c0:2 * C + c0 + HD]
                    e = jnp.exp(_mmt(q, k) * SCALE + lb)
                    ve = jnp.concatenate([v, ones], axis=1)
                    oe = _mm(e.astype(bf16), ve)
                    den = oe[:, HD:HD + 1] + npad
                    ao_s[r0:r0 + NS, c0:c0 + HD] = (
                        oe[:, :HD] / den).astype(bf16)
            out = _mmt(ao_s[...], lpw[0].astype(bf16)) + lnp[0, 4:5]
            o2 = out.reshape(2, NP, C)
            h_s[:, :NS, :] = h_s[:, :NS, :] + o2[:, :NS, :]

        @pl.when(~even)
        def _():
            gb = cm_ref[NS:NS + 1, :NP]
            ones = jnp.ones((NP, 1), bf16)
            for b in range(2):
                r0 = b * NP
                for hh in range(HEADS):
                    c0 = hh * HD
                    q = qkv_s[r0:r0 + NP, c0:c0 + HD]
                    k = qkv_s[r0:r0 + NP, C + c0:C + c0 + HD]
                    v = qkv_s[r0:r0 + NP, 2 * C + c0:2 * C + c0 + HD]
                    e = jnp.exp(_mmt(q, k) * SCALE + gb)
                    ve = jnp.concatenate([v, ones], axis=1)
                    oe = _mm(e.astype(bf16), ve)
                    ao_s[r0:r0 + NP, c0:c0 + HD] = (
                        oe[:, :HD] / oe[:, HD:HD + 1]).astype(bf16)
            out = _mmt(ao_s[...], gow[0].astype(bf16)) + lnp[0, 4:5]
            h_s[...] = h_s[...] + out.reshape(2, NP, C)

        ln_s[...] = _ln(h_s[...].reshape(MH, C), lnp[0, 2:3], lnp[0, 3:4]
                        ).astype(bf16)

    @pl.when(t >= 3)
    def _():
        hcol = _mmt(ln_s[...], f1w[0].astype(bf16)) + f1b[0]
        lane = (t - 3) * CT + jax.lax.broadcasted_iota(jnp.int32, (1, CT), 1)
        valid = lane < CH
        hcol = jnp.where(valid, _gelu(hcol), 0.0).astype(bf16)
        f2wv = jnp.where(valid, f2w[0].astype(bf16), jnp.bfloat16(0.0))
        part = _mmt(hcol, f2wv).reshape(2, NP, C)

        @pl.when(t < NSTG - 1)
        def _():
            h_s[...] = h_s[...] + part

        @pl.when(t == NSTG - 1)
        def _():
            fin = h_s[...].reshape(MH, C) + part.reshape(MH, C) + lnp[0, 5:6]
            h_s[...] = fin.reshape(2, NP, C)

            @pl.when(l == DEPTH - 1)
            def _():
                o_ref[...] = _ln(fin, fn_ref[0:1], fn_ref[1:2]
                                 ).reshape(2, NP, C)


def _mega(h, lnp, lqw, lpw, gqw, gqb3, gow, f1w, f1b3, f2w, fn, cm):
    qc = lambda t: jnp.minimum(t, 1)
    mc = lambda t: jnp.clip(t - 3, 0, MT - 1)
    e_ = lambda l: l // 2
    even_ = lambda l: (l % 2) == 0
    lq_c = lambda l, t: jnp.where(even_(l), qc(t), 1)
    gq_c = lambda l, t: jnp.where(even_(l), 1, qc(t))

    in_specs = [
        pl.BlockSpec((2, NP, C), lambda c, l, t: (c, 0, 0)),
        pl.BlockSpec((1, 6, C), lambda c, l, t: (l, 0, 0)),
        pl.BlockSpec((1, QC, C), lambda c, l, t: (e_(l), lq_c(l, t), 0)),
        pl.BlockSpec((1, C, C), lambda c, l, t: (e_(l), 0, 0)),
        pl.BlockSpec((1, QC, C), lambda c, l, t: (e_(l), gq_c(l, t), 0)),
        pl.BlockSpec((1, 1, QC), lambda c, l, t: (e_(l), 0, gq_c(l, t))),
        pl.BlockSpec((1, C, C), lambda c, l, t: (e_(l), 0, 0)),
        pl.BlockSpec((1, CT, C), lambda c, l, t: (l, mc(t), 0)),
        pl.BlockSpec((1, 1, CT), lambda c, l, t: (l, 0, mc(t))),
        pl.BlockSpec((1, C, CT), lambda c, l, t: (l, 0, mc(t))),
        pl.BlockSpec((2, C), lambda c, l, t: (0, 0)),
        pl.BlockSpec((NP, 3 * 128), lambda c, l, t: (0, 0)),
    ]
    return pl.pallas_call(
        _mega_body,
        grid=(2, DEPTH, NSTG),
        in_specs=in_specs,
        out_specs=pl.BlockSpec((2, NP, C), lambda c, l, t: (c, 0, 0)),
        out_shape=jax.ShapeDtypeStruct((4, NP, C), jnp.float32),
        scratch_shapes=[pltpu.VMEM((2, NP, C), jnp.float32),
                        pltpu.VMEM((MH, C), jnp.bfloat16),
                        pltpu.VMEM((MH, 3 * C), jnp.bfloat16),
                        pltpu.VMEM((MH, C), jnp.bfloat16)],
        compiler_params=pltpu.CompilerParams(
            dimension_semantics=("parallel", "arbitrary", "arbitrary"),
            vmem_limit_bytes=_VMEM_LIMIT),
        name="d4rt_layers",
    )(h, lnp, lqw, lpw, gqw, gqb3, gow, f1w, f1b3, f2w, fn, cm)


def kernel(x, aspect_ratio, conv_w, conv_b, pe_ln_w, pe_ln_b, ar_token,
           t_pos, s_pos, n1_w, n1_b, n2_w, n2_b, loc_qkv_w, loc_proj_w,
           loc_proj_b, glb_in_w, glb_in_b, glb_out_w, glb_out_b,
           fc1_w, fc1_b, fc2_w, fc2_b, fn_w, fn_b):
    B = x.shape[0]
    f32 = jnp.float32

    xp = x.transpose(0, 2, 1, 3, 4).reshape(B, 3, 2, 16, 16, 16, 16)
    patches = xp.transpose(0, 3, 5, 1, 2, 4, 6).reshape(B, NS, PK)
    wpe = conv_w.reshape(C, PK)
    pos = (t_pos[0, 0][None, :] + s_pos[0]).astype(f32)

    row = lambda a: a.reshape(1, -1).astype(f32)
    tokens = _embed(patches, wpe, row(conv_b), row(pe_ln_w), row(pe_ln_b), pos)

    ar_rows = ar_token * (1.0 + aspect_ratio[:, None, None] * 0.1)
    h = jnp.concatenate(
        [tokens, ar_rows.astype(f32), jnp.zeros((B, NP - NT, C), f32)], axis=1)

    projb = jnp.stack([loc_proj_b, glb_out_b], axis=1).reshape(DEPTH, C)
    lnp = jnp.stack([n1_w, n1_b, n2_w, n2_b, projb, fc2_b], axis=1)
    fn = jnp.stack([fn_w, fn_b])
    cm = jnp.asarray(_CMASK_NP)

    out = _mega(h, lnp, loc_qkv_w, loc_proj_w, glb_in_w,
                glb_in_b.reshape(6, 1, 3 * C), glb_out_w,
                fc1_w, fc1_b.reshape(DEPTH, 1, CH), fc2_w, fn, cm)
    return out[:, :NT]
